# baseline (speedup 1.0000x reference)
import jax
import jax.numpy as jnp
from jax import lax
from jax.experimental import pallas as pl
from jax.experimental.pallas import tpu as pltpu
from jax.experimental.pallas import tpu_sc as plsc

N_ROWS = 128
N_COLS = 32768
RANK_IDX = int(0.9 * N_COLS)
K_TOP = N_COLS - RANK_IDX
LANES = 16
CHUNKS = N_COLS // LANES
NC, NS = 2, 16
NW = NC * NS
ROWS_PER_W = N_ROWS // NW
U = 2

_MESH = plsc.VectorSubcoreMesh(
    core_axis_name="c", subcore_axis_name="s", num_cores=NC, num_subcores=NS
)

_MININT = -2147483648


def _keys_of(x16):
    xi = plsc.bitcast(x16, jnp.int32)
    m = lax.shift_right_arithmetic(xi, 31)
    return xi ^ (m | _MININT)


def _srl(x, n):
    return lax.shift_right_logical(x, n)


def _tec_body(x_hbm, out_hbm, xbuf, cbuf, hist):
    cid = lax.axis_index("c")
    sid = lax.axis_index("s")
    wid = sid * NC + cid

    ones = jnp.full((LANES,), 1, jnp.int32)
    zeros = jnp.zeros((LANES,), jnp.int32)
    iota = lax.iota(jnp.int32, LANES)

    def zero_hist():
        for j in range(256 // LANES):
            hist[pl.ds(j * LANES, LANES)] = zeros

    def pick_byte(r_cur):
        above = jnp.int32(0)
        nge = jnp.int32(0)
        cab = jnp.int32(0)
        for j in range(15, -1, -1):
            v = hist[pl.ds(j * LANES, LANES)]
            cs = jnp.cumsum(lax.rev(v, (0,)))
            C = lax.rev(cs, (0,)) + above
            ge = C >= r_cur
            nge = nge + jnp.sum(ge.astype(jnp.int32))
            cab = jnp.maximum(cab, jnp.max(jnp.where(ge, 0, C)))
            above = above + jnp.sum(v)
        return nge - 1, r_cur - cab

    def do_row(row, _):
        pltpu.sync_copy(x_hbm.at[row], xbuf)

        zero_hist()

        @plsc.parallel_loop(0, CHUNKS, unroll=16)
        def l0(i):
            key = _keys_of(xbuf[pl.ds(i * LANES, LANES)])
            b = _srl(key, 24)
            plsc.addupdate_scatter(hist, [b], ones)
        B, r_cur = pick_byte(jnp.int32(K_TOP))
        p8 = B

        @plsc.parallel_loop(0, CHUNKS, unroll=8, carry=zeros)
        def l1(i, w):
            key = _keys_of(xbuf[pl.ds(i * LANES, LANES)])
            match = _srl(key, 24) == p8
            pos = w + jnp.cumsum(match.astype(jnp.int32)) - 1
            plsc.store_scatter(cbuf, [pos], key, mask=match)
            return w + plsc.all_reduce_population_count(match)

        n1 = jnp.max(l1)
        zero_hist()

        @plsc.parallel_loop(0, (n1 + LANES - 1) // LANES, unroll=4)
        def l1h(i):
            base = i * LANES
            key = cbuf[pl.ds(base, LANES)]
            b = _srl(key, 16) & 0xFF
            plsc.addupdate_scatter(hist, [b], ones, mask=iota < (n1 - base))

        B, r_cur = pick_byte(r_cur)
        p16 = (p8 << 8) | B

        zero_hist()

        def l2(i, w):
            base = i * LANES
            key = cbuf[pl.ds(base, LANES)]
            match = (_srl(key, 16) == p16) & (iota < (n1 - base))
            b = _srl(key, 8) & 0xFF
            plsc.addupdate_scatter(hist, [b], ones, mask=match)
            pos = w + jnp.cumsum(match.astype(jnp.int32)) - 1
            plsc.store_scatter(cbuf, [pos], key, mask=match)
            return w + plsc.all_reduce_population_count(match)

        n2v = lax.fori_loop(0, (n1 + LANES - 1) // LANES, l2, zeros)
        n2 = jnp.max(n2v)
        B, r_cur = pick_byte(r_cur)
        p24 = (p16 << 8) | B

        zero_hist()

        def l3(i, carry):
            base = i * LANES
            key = cbuf[pl.ds(base, LANES)]
            match = (_srl(key, 8) == p24) & (iota < (n2 - base))
            b = key & 0xFF
            plsc.addupdate_scatter(hist, [b], ones, mask=match)
            return carry

        lax.fori_loop(0, (n2 + LANES - 1) // LANES, l3, 0)
        B, _ = pick_byte(r_cur)
        kthr = lax.shift_left(p24, 8) | B

        kv = jnp.broadcast_to(kthr, (LANES,))
        tb = jnp.where(kv < 0, kv ^ _MININT, ~kv)
        theta = plsc.bitcast(tb, jnp.float32)

        @plsc.parallel_loop(0, CHUNKS, unroll=16)
        def mk(i):
            sl = pl.ds(i * LANES, LANES)
            x16 = xbuf[sl]
            xbuf[sl] = jnp.where(x16 >= theta, x16, 0.0)
        pltpu.sync_copy(xbuf, out_hbm.at[row])
        return _

    lax.fori_loop(wid * ROWS_PER_W, (wid + 1) * ROWS_PER_W, do_row, 0)


_ksparse_sc = pl.kernel(
    _tec_body,
    out_type=jax.ShapeDtypeStruct((N_ROWS, N_COLS), jnp.float32),
    mesh=_MESH,
    scratch_types=[
        pltpu.VMEM((N_COLS,), jnp.float32),
        pltpu.VMEM((N_COLS,), jnp.int32),
        pltpu.VMEM((256,), jnp.int32),
    ],
    compiler_params=pltpu.CompilerParams(needs_layout_passes=False),
    name="ksparse_radix_select_sc",
)


def kernel(X):
    return _ksparse_sc(X)


# speedup vs baseline: 1.0032x; 1.0032x over previous
import jax
import jax.numpy as jnp
from jax import lax
from jax.experimental import pallas as pl
from jax.experimental.pallas import tpu as pltpu
from jax.experimental.pallas import tpu_sc as plsc

N_ROWS = 128
N_COLS = 32768
RANK_IDX = int(0.9 * N_COLS)
K_TOP = N_COLS - RANK_IDX
LANES = 16
CHUNKS = N_COLS // LANES
NC, NS = 2, 16
NW = NC * NS
ROWS_PER_W = N_ROWS // NW

_MESH = plsc.VectorSubcoreMesh(
    core_axis_name="c", subcore_axis_name="s", num_cores=NC, num_subcores=NS
)

_MININT = -2147483648


def _keys_of(x16):
    xi = plsc.bitcast(x16, jnp.int32)
    m = lax.shift_right_arithmetic(xi, 31)
    return xi ^ (m | _MININT)


def _srl(x, n):
    return lax.shift_right_logical(x, n)


def _tec_body(x_hbm, th_hbm, xbuf, cbuf, hist, tbuf, sem):
    cid = lax.axis_index("c")
    sid = lax.axis_index("s")
    wid = sid * NC + cid
    base = wid * ROWS_PER_W

    ones = jnp.full((LANES,), 1, jnp.int32)
    zeros = jnp.zeros((LANES,), jnp.int32)
    iota = lax.iota(jnp.int32, LANES)

    def zero_hist():
        for j in range(256 // LANES):
            hist[pl.ds(j * LANES, LANES)] = zeros

    def pick_byte(r_cur):
        above = jnp.int32(0)
        nge = jnp.int32(0)
        cab = jnp.int32(0)
        for j in range(15, -1, -1):
            v = hist[pl.ds(j * LANES, LANES)]
            cs = jnp.cumsum(lax.rev(v, (0,)))
            C = lax.rev(cs, (0,)) + above
            ge = C >= r_cur
            nge = nge + jnp.sum(ge.astype(jnp.int32))
            cab = jnp.maximum(cab, jnp.max(jnp.where(ge, 0, C)))
            above = above + jnp.sum(v)
        return nge - 1, r_cur - cab

    pltpu.async_copy(x_hbm.at[base], xbuf.at[pl.ds(0, N_COLS)], sem)

    def do_row(row, _):
        half = (row - base) & 1
        off = half * N_COLS

        @pl.when(row + 1 < base + ROWS_PER_W)
        def _prefetch():
            pltpu.async_copy(
                x_hbm.at[row + 1], xbuf.at[pl.ds((1 - half) * N_COLS, N_COLS)],
                sem,
            )

        pltpu.make_async_copy(
            x_hbm.at[row], xbuf.at[pl.ds(off, N_COLS)], sem
        ).wait()

        zero_hist()

        @plsc.parallel_loop(0, CHUNKS, unroll=16)
        def l0(i):
            key = _keys_of(xbuf[pl.ds(off + i * LANES, LANES)])
            b = _srl(key, 24)
            plsc.addupdate_scatter(hist, [b], ones)
        B, r_cur = pick_byte(jnp.int32(K_TOP))
        p8 = B

        @plsc.parallel_loop(0, CHUNKS, unroll=8, carry=zeros)
        def l1(i, w):
            key = _keys_of(xbuf[pl.ds(off + i * LANES, LANES)])
            match = _srl(key, 24) == p8
            pos = w + jnp.cumsum(match.astype(jnp.int32)) - 1
            plsc.store_scatter(cbuf, [pos], key, mask=match)
            return w + plsc.all_reduce_population_count(match)

        n1 = jnp.max(l1)
        zero_hist()

        @plsc.parallel_loop(0, (n1 + LANES - 1) // LANES, unroll=4)
        def l1h(i):
            b2 = i * LANES
            key = cbuf[pl.ds(b2, LANES)]
            b = _srl(key, 16) & 0xFF
            plsc.addupdate_scatter(hist, [b], ones, mask=iota < (n1 - b2))

        B, r_cur = pick_byte(r_cur)
        p16 = (p8 << 8) | B

        zero_hist()

        def l2(i, w):
            b2 = i * LANES
            key = cbuf[pl.ds(b2, LANES)]
            match = (_srl(key, 16) == p16) & (iota < (n1 - b2))
            b = _srl(key, 8) & 0xFF
            plsc.addupdate_scatter(hist, [b], ones, mask=match)
            pos = w + jnp.cumsum(match.astype(jnp.int32)) - 1
            plsc.store_scatter(cbuf, [pos], key, mask=match)
            return w + plsc.all_reduce_population_count(match)

        n2v = lax.fori_loop(0, (n1 + LANES - 1) // LANES, l2, zeros)
        n2 = jnp.max(n2v)
        B, r_cur = pick_byte(r_cur)
        p24 = (p16 << 8) | B

        zero_hist()

        def l3(i, carry):
            b2 = i * LANES
            key = cbuf[pl.ds(b2, LANES)]
            match = (_srl(key, 8) == p24) & (iota < (n2 - b2))
            b = key & 0xFF
            plsc.addupdate_scatter(hist, [b], ones, mask=match)
            return carry

        lax.fori_loop(0, (n2 + LANES - 1) // LANES, l3, 0)
        B, _ = pick_byte(r_cur)
        kthr = lax.shift_left(p24, 8) | B

        kv = jnp.broadcast_to(kthr, (LANES,))
        tb = jnp.where(kv < 0, kv ^ _MININT, ~kv)
        theta = plsc.bitcast(tb, jnp.float32)
        tbuf[pl.ds((row - base) * LANES, LANES)] = theta
        return _

    lax.fori_loop(base, base + ROWS_PER_W, do_row, 0)
    pltpu.sync_copy(tbuf, th_hbm.at[pl.ds(base * LANES, ROWS_PER_W * LANES)])


_ksparse_theta_sc = pl.kernel(
    _tec_body,
    out_type=jax.ShapeDtypeStruct((N_ROWS * LANES,), jnp.float32),
    mesh=_MESH,
    scratch_types=[
        pltpu.VMEM((2 * N_COLS,), jnp.float32),
        pltpu.VMEM((N_COLS,), jnp.int32),
        pltpu.VMEM((256,), jnp.int32),
        pltpu.VMEM((ROWS_PER_W * LANES,), jnp.float32),
        pltpu.SemaphoreType.DMA,
    ],
    compiler_params=pltpu.CompilerParams(needs_layout_passes=False),
    name="ksparse_radix_select_sc",
)

_MASK_BC = 4096


def _mask_body(x_ref, th_ref, o_ref):
    th = th_ref[:, 0:1]
    x = x_ref[...]
    o_ref[...] = jnp.where(x >= th, x, jnp.float32(0.0))


_mask_tc = pl.pallas_call(
    _mask_body,
    grid=(N_COLS // _MASK_BC,),
    in_specs=[
        pl.BlockSpec((N_ROWS, _MASK_BC), lambda i: (0, i)),
        pl.BlockSpec((N_ROWS, LANES), lambda i: (0, 0)),
    ],
    out_specs=pl.BlockSpec((N_ROWS, _MASK_BC), lambda i: (0, i)),
    out_shape=jax.ShapeDtypeStruct((N_ROWS, N_COLS), jnp.float32),
)


def kernel(X):
    thetas = _ksparse_theta_sc(X).reshape(N_ROWS, LANES)
    return _mask_tc(X, thetas)


# speedup vs baseline: 1.2518x; 1.2478x over previous
import jax
import jax.numpy as jnp
from jax import lax
from jax.experimental import pallas as pl
from jax.experimental.pallas import tpu as pltpu
from jax.experimental.pallas import tpu_sc as plsc

N_ROWS = 128
N_COLS = 32768
RANK_IDX = int(0.9 * N_COLS)
K_TOP = N_COLS - RANK_IDX
LANES = 16
CHUNKS = N_COLS // LANES
NC, NS = 2, 16
NW = NC * NS
ROWS_PER_W = N_ROWS // NW

_MESH = plsc.VectorSubcoreMesh(
    core_axis_name="c", subcore_axis_name="s", num_cores=NC, num_subcores=NS
)

_MININT = -2147483648


def _keys_of(x16):
    xi = plsc.bitcast(x16, jnp.int32)
    m = lax.shift_right_arithmetic(xi, 31)
    return xi ^ (m | _MININT)


def _srl(x, n):
    return lax.shift_right_logical(x, n)


def _tec_body(x_hbm, th_hbm, xbuf, cbuf, hist, tbuf, sem):
    cid = lax.axis_index("c")
    sid = lax.axis_index("s")
    wid = sid * NC + cid
    base = wid * ROWS_PER_W

    ones = jnp.full((LANES,), 1, jnp.int32)
    zeros = jnp.zeros((LANES,), jnp.int32)
    iota = lax.iota(jnp.int32, LANES)

    def zero_hist():
        for j in range(256 // LANES):
            hist[pl.ds(j * LANES, LANES)] = zeros

    def pick_byte(r_cur):
        above = jnp.int32(0)
        nge = jnp.int32(0)
        cab = jnp.int32(0)
        for j in range(15, -1, -1):
            v = hist[pl.ds(j * LANES, LANES)]
            cs = jnp.cumsum(lax.rev(v, (0,)))
            C = lax.rev(cs, (0,)) + above
            ge = C >= r_cur
            nge = nge + jnp.sum(ge.astype(jnp.int32))
            cab = jnp.maximum(cab, jnp.max(jnp.where(ge, 0, C)))
            above = above + jnp.sum(v)
        return nge - 1, r_cur - cab

    pltpu.async_copy(x_hbm.at[base], xbuf.at[pl.ds(0, N_COLS)], sem)

    def do_row(row, _):
        half = (row - base) & 1
        off = half * N_COLS

        @pl.when(row + 1 < base + ROWS_PER_W)
        def _prefetch():
            pltpu.async_copy(
                x_hbm.at[row + 1], xbuf.at[pl.ds((1 - half) * N_COLS, N_COLS)],
                sem,
            )

        pltpu.make_async_copy(
            x_hbm.at[row], xbuf.at[pl.ds(off, N_COLS)], sem
        ).wait()

        zero_hist()

        @plsc.parallel_loop(0, CHUNKS, unroll=16)
        def l0(i):
            key = _keys_of(xbuf[pl.ds(off + i * LANES, LANES)])
            b = _srl(key, 24)
            plsc.addupdate_scatter(hist, [b], ones)
        B, r_cur = pick_byte(jnp.int32(K_TOP))
        p8 = B

        @plsc.parallel_loop(0, CHUNKS, unroll=8, carry=zeros)
        def l1(i, w):
            key = _keys_of(xbuf[pl.ds(off + i * LANES, LANES)])
            match = _srl(key, 24) == p8
            pos = w + jnp.cumsum(match.astype(jnp.int32)) - 1
            plsc.store_scatter(cbuf, [pos], key, mask=match)
            return w + plsc.all_reduce_population_count(match)

        n1 = jnp.max(l1)
        zero_hist()

        @plsc.parallel_loop(0, (n1 + LANES - 1) // LANES, unroll=8)
        def l1h(i):
            b2 = i * LANES
            key = cbuf[pl.ds(b2, LANES)]
            b = _srl(key, 16) & 0xFF
            plsc.addupdate_scatter(hist, [b], ones, mask=iota < (n1 - b2))

        B, r_cur = pick_byte(r_cur)
        p16 = (p8 << 8) | B

        zero_hist()

        @plsc.parallel_loop(0, (n1 + LANES - 1) // LANES, unroll=4,
                            carry=zeros)
        def l2(i, w):
            b2 = i * LANES
            key = cbuf[pl.ds(b2, LANES)]
            match = (_srl(key, 16) == p16) & (iota < (n1 - b2))
            b = _srl(key, 8) & 0xFF
            plsc.addupdate_scatter(hist, [b], ones, mask=match)
            pos = w + jnp.cumsum(match.astype(jnp.int32)) - 1
            plsc.store_scatter(cbuf, [pos], key, mask=match)
            return w + plsc.all_reduce_population_count(match)

        n2 = jnp.max(l2)
        B, r_cur = pick_byte(r_cur)
        p24 = (p16 << 8) | B

        zero_hist()

        @plsc.parallel_loop(0, (n2 + LANES - 1) // LANES, unroll=4)
        def l3(i):
            b2 = i * LANES
            key = cbuf[pl.ds(b2, LANES)]
            match = (_srl(key, 8) == p24) & (iota < (n2 - b2))
            b = key & 0xFF
            plsc.addupdate_scatter(hist, [b], ones, mask=match)
        B, _ = pick_byte(r_cur)
        kthr = lax.shift_left(p24, 8) | B

        kv = jnp.broadcast_to(kthr, (LANES,))
        tb = jnp.where(kv < 0, kv ^ _MININT, ~kv)
        theta = plsc.bitcast(tb, jnp.float32)
        tbuf[pl.ds((row - base) * LANES, LANES)] = theta
        return _

    lax.fori_loop(base, base + ROWS_PER_W, do_row, 0)
    pltpu.sync_copy(tbuf, th_hbm.at[pl.ds(base * LANES, ROWS_PER_W * LANES)])


_ksparse_theta_sc = pl.kernel(
    _tec_body,
    out_type=jax.ShapeDtypeStruct((N_ROWS * LANES,), jnp.float32),
    mesh=_MESH,
    scratch_types=[
        pltpu.VMEM((2 * N_COLS,), jnp.float32),
        pltpu.VMEM((N_COLS,), jnp.int32),
        pltpu.VMEM((256,), jnp.int32),
        pltpu.VMEM((ROWS_PER_W * LANES,), jnp.float32),
        pltpu.SemaphoreType.DMA,
    ],
    compiler_params=pltpu.CompilerParams(needs_layout_passes=False),
    name="ksparse_radix_select_sc",
)

_MASK_BC = 4096


def _mask_body(x_ref, th_ref, o_ref):
    th = th_ref[:, 0:1]
    x = x_ref[...]
    o_ref[...] = jnp.where(x >= th, x, jnp.float32(0.0))


_mask_tc = pl.pallas_call(
    _mask_body,
    grid=(N_COLS // _MASK_BC,),
    in_specs=[
        pl.BlockSpec((N_ROWS, _MASK_BC), lambda i: (0, i)),
        pl.BlockSpec((N_ROWS, LANES), lambda i: (0, 0)),
    ],
    out_specs=pl.BlockSpec((N_ROWS, _MASK_BC), lambda i: (0, i)),
    out_shape=jax.ShapeDtypeStruct((N_ROWS, N_COLS), jnp.float32),
)


def kernel(X):
    thetas = _ksparse_theta_sc(X).reshape(N_ROWS, LANES)
    return _mask_tc(X, thetas)
